# baseline (speedup 1.0000x reference)
import functools

import jax
import jax.numpy as jnp
from jax import lax
from jax.experimental import pallas as pl
from jax.experimental.pallas import tpu as pltpu
from jax.experimental.pallas import tpu_sc as plsc

BLANK = 0
EMBED = 64
NUM_CORES = 2
NUM_SUBCORES = 16
LANES = 16
NUM_WORKERS = NUM_CORES * NUM_SUBCORES

CHUNK = 128
D = 6
L = 5


def _make_lookup(total_lines):
    per_w = total_lines // NUM_WORKERS
    n_chunks = per_w // CHUNK
    assert n_chunks * CHUNK == per_w
    assert L < n_chunks and D > L
    mesh = plsc.VectorSubcoreMesh(core_axis_name="c", subcore_axis_name="s")

    @functools.partial(
        pl.kernel,
        out_type=jax.ShapeDtypeStruct((total_lines, 2 * EMBED), jnp.float32),
        mesh=mesh,
        scratch_types=[
            pltpu.VMEM((2 * per_w,), jnp.int32),
            pltpu.VMEM((per_w,), jnp.int32),
            pltpu.VMEM((per_w,), jnp.int32),
            pltpu.VMEM((D * CHUNK, EMBED), jnp.float32),
            pltpu.VMEM((D * CHUNK, EMBED), jnp.float32),
            pltpu.SemaphoreType.DMA((D,)),
            pltpu.SemaphoreType.DMA((D,)),
            pltpu.SemaphoreType.DMA((D,)),
            pltpu.SemaphoreType.DMA((D,)),
        ],
        compiler_params=pltpu.CompilerParams(use_tc_tiling_on_sc=False),
    )
    def lookup(
        idx_hbm, table_hbm, out_hbm,
        idx2_v, idx_e_v, idx_o_v, e_rows, o_rows, gsem_e, gsem_o, ssem_e, ssem_o,
    ):
        wid = lax.axis_index("s") * NUM_CORES + lax.axis_index("c")
        base = wid * per_w
        pltpu.sync_copy(idx_hbm.at[pl.ds(2 * base, 2 * per_w)], idx2_v)

        lanes = lax.broadcasted_iota(jnp.int32, (LANES,), 0)
        zeros = jnp.zeros((LANES,), jnp.float32)
        perm_e = (2 * lanes) % LANES
        perm_o = (2 * lanes + 1) % LANES
        lo_half = lanes < (LANES // 2)

        def deinterleave(j):
            for g in range(CHUNK // LANES):
                off = 2 * j * CHUNK + 2 * g * LANES
                v0 = idx2_v[pl.ds(off, LANES)]
                v1 = idx2_v[pl.ds(off + LANES, LANES)]
                ev = jnp.where(lo_half, jnp.take(v0, perm_e), jnp.take(v1, perm_e))
                ov = jnp.where(lo_half, jnp.take(v0, perm_o), jnp.take(v1, perm_o))
                dst = j * CHUNK + g * LANES
                idx_e_v[pl.ds(dst, LANES)] = ev
                idx_o_v[pl.ds(dst, LANES)] = ov

        def gather(i, slot, idx_v, rows_v, gsem):
            return pltpu.make_async_copy(
                table_hbm.at[idx_v.at[pl.ds(i * CHUNK, CHUNK)]],
                rows_v.at[pl.ds(slot * CHUNK, CHUNK)],
                gsem.at[slot],
            )

        def store(i, slot, rows_v, ssem, lane_off):
            return pltpu.make_async_copy(
                rows_v.at[pl.ds(slot * CHUNK, CHUNK)],
                out_hbm.at[pl.ds(base + i * CHUNK, CHUNK),
                           pl.ds(lane_off, EMBED)],
                ssem.at[slot],
            )

        def store_e(i, slot):
            return store(i, slot, e_rows, ssem_e, 0)

        def store_o(i, slot):
            return store(i, slot, o_rows, ssem_o, EMBED)

        def fixup(slot, i, idx_v, rows_v):
            buf_off = slot * CHUNK
            chunk_off = i * CHUNK
            m_any = None
            for g in range(CHUNK // LANES):
                iv = idx_v[pl.ds(chunk_off + g * LANES, LANES)]
                m = iv == BLANK
                m_any = m if m_any is None else jnp.logical_or(m_any, m)
            v = jnp.where(m_any, jnp.int32(1), jnp.int32(0))
            for sh in (8, 4, 2, 1):
                v = v | jnp.take(v, (lanes + sh) % LANES)

            @pl.when(v[0] > 0)
            def _fix():
                def group_body(g, carry):
                    iv = idx_v[pl.ds(chunk_off + g * LANES, LANES)]
                    for l in range(LANES):
                        row = g * LANES + l

                        @pl.when(iv[l] == BLANK)
                        def _zero_row(row=row):
                            for c in range(EMBED // LANES):
                                rows_v[
                                    buf_off + row, pl.ds(c * LANES, LANES)
                                ] = zeros

                    return carry

                lax.fori_loop(0, CHUNK // LANES, group_body, 0)

        for i in range(L):
            deinterleave(i)
            gather(i, i, idx_e_v, e_rows, gsem_e).start()
            gather(i, i, idx_o_v, o_rows, gsem_o).start()

        def body(i, carry):
            slot = i % D
            gather(i, slot, idx_e_v, e_rows, gsem_e).wait()
            gather(i, slot, idx_o_v, o_rows, gsem_o).wait()
            fixup(slot, i, idx_e_v, e_rows)
            fixup(slot, i, idx_o_v, o_rows)
            store_e(i, slot).start()

            @pl.when(i >= 1)
            def _odd_prev():
                store_o(i - 1, (i - 1) % D).start()

            j = i + L
            sj = j % D

            @pl.when(jnp.logical_and(j < n_chunks, j >= D))
            def _reuse():
                store_e(j - D, sj).wait()
                store_o(j - D, sj).wait()

            @pl.when(j < n_chunks)
            def _next():
                deinterleave(j)
                gather(j, sj, idx_e_v, e_rows, gsem_e).start()
                gather(j, sj, idx_o_v, o_rows, gsem_o).start()

            return carry

        lax.fori_loop(0, n_chunks, body, 0)

        store_o(n_chunks - 1, (n_chunks - 1) % D).start()
        for i in range(n_chunks - D, n_chunks):
            store_e(i, i % D).wait()
            store_o(i, i % D).wait()

    return lookup


_LOOKUP_CACHE = {}


def kernel(history, table):
    b, u, h = history.shape
    assert h == 2
    lines = b * u
    if lines not in _LOOKUP_CACHE:
        _LOOKUP_CACHE[lines] = _make_lookup(lines)
    idx = history.reshape(lines * 2)
    out = _LOOKUP_CACHE[lines](idx, table)
    return out.reshape(b, u, 2 * EMBED)
